# baseline (speedup 1.0000x reference)
import functools
import math

import jax
import jax.numpy as jnp
from jax import lax
from jax.experimental import pallas as pl
from jax.experimental.pallas import tpu as pltpu
from jax.experimental.pallas import tpu_sc as plsc

NUM_SAMPLED = 100
NUM_CLASSES = 100000
DIM = 64
BATCH = 4096
S_PAD = 128
NC = 2
NS = 16
NW = NC * NS
CHUNK = BATCH // NW
L = 16

_INV_LOG_RANGE = 1.0 / math.log(NUM_CLASSES + 1.0)


def _zero_tail(ref):
    for i in range(NUM_SAMPLED // L, S_PAD // L):
        sl = pl.ds(i * L, L)
        lane = lax.broadcasted_iota(jnp.int32, (L,), 0) + i * L
        ref[sl] = jnp.where(lane < NUM_SAMPLED, ref[sl], 0.0)


def _sc_body(item_flat, user_hbm, lab_hbm, sid_hbm, bias_hbm,
             true_out, sw_out, sb_out,
             lab_v, idx_v, user_v, g_v, acc_v, sid_v, sg_v):
    wid = lax.axis_index("s") * NC + lax.axis_index("c")
    base = wid * CHUNK

    pltpu.sync_copy(lab_hbm.at[pl.ds(base, CHUNK)], lab_v)
    pltpu.sync_copy(user_hbm.at[:, pl.ds(base, CHUNK)], user_v)
    pltpu.sync_copy(bias_hbm.at[lab_v], acc_v)

    @pl.loop(0, DIM)
    def _(d):
        off = d * NUM_CLASSES
        for i in range(CHUNK // L):
            sl = pl.ds(i * L, L)
            idx_v[sl] = lab_v[sl] + off
        pltpu.sync_copy(item_flat.at[idx_v], g_v)
        for i in range(CHUNK // L):
            sl = pl.ds(i * L, L)
            acc_v[sl] = acc_v[sl] + user_v[d, sl] * g_v[sl]

    pltpu.sync_copy(acc_v, true_out.at[pl.ds(base, CHUNK)])

    pltpu.sync_copy(sid_hbm, sid_v)
    for r in range(DIM // NW):
        d = wid * (DIM // NW) + r
        off = d * NUM_CLASSES
        for i in range(S_PAD // L):
            sl = pl.ds(i * L, L)
            idx_v[sl] = sid_v[sl] + off
        pltpu.sync_copy(item_flat.at[idx_v], sg_v)
        _zero_tail(sg_v)
        pltpu.sync_copy(sg_v, sw_out.at[d])

    @pl.when(wid == 0)
    def _():
        pltpu.sync_copy(bias_hbm.at[sid_v], sg_v)
        _zero_tail(sg_v)
        pltpu.sync_copy(sg_v, sb_out)


@jax.jit
def _sc_gather(item_flat, user_emb, labels, sampled_pad, bias):
    mesh = plsc.VectorSubcoreMesh(core_axis_name="c", subcore_axis_name="s")
    f = pl.kernel(
        _sc_body,
        out_type=(
            jax.ShapeDtypeStruct((BATCH,), jnp.float32),
            jax.ShapeDtypeStruct((DIM, S_PAD), jnp.float32),
            jax.ShapeDtypeStruct((S_PAD,), jnp.float32),
        ),
        mesh=mesh,
        scratch_types=[
            pltpu.VMEM((CHUNK,), jnp.int32),
            pltpu.VMEM((CHUNK,), jnp.int32),
            pltpu.VMEM((DIM, CHUNK), jnp.float32),
            pltpu.VMEM((CHUNK,), jnp.float32),
            pltpu.VMEM((CHUNK,), jnp.float32),
            pltpu.VMEM((S_PAD,), jnp.int32),
            pltpu.VMEM((S_PAD,), jnp.float32),
        ],
    )
    return f(item_flat, user_emb, labels, sampled_pad, bias)


def _tc_body(user_ref, sw_ref, td_ref, lab_ref, sid_ref, sb_ref, corr_ref,
             out_ref):
    x = user_ref[...]
    w = sw_ref[...]
    sl = lax.dot_general(x, w, (((0,), (0,)), ((), ())),
                         preferred_element_type=jnp.float32,
                         precision=lax.Precision.HIGHEST)
    sl = sl + sb_ref[...] - corr_ref[...]

    lab = lab_ref[...]
    sid = sid_ref[...]
    hits = sid == lab
    sl = jnp.where(hits, sl - 1e9, sl)
    col = lax.broadcasted_iota(jnp.int32, (1, S_PAD), 1)
    sl = jnp.where(col < NUM_SAMPLED, sl, -1e30)

    labf = lab.astype(jnp.float32)
    q_true = jnp.log((labf + 2.0) / (labf + 1.0)) * _INV_LOG_RANGE
    q = q_true
    l1p = -(q * (1.0 + q * (0.5 + q * (1.0 / 3.0 + q * (0.25 + q * 0.2)))))
    xx = NUM_SAMPLED * l1p
    small = xx > -0.2
    series = xx * (1.0 + xx * (0.5 + xx * (1.0 / 6.0 + xx * (1.0 / 24.0))))
    exp_true = -jnp.where(small, series, jnp.exp(xx) - 1.0)
    tl = td_ref[...] - jnp.log(exp_true)

    m = jnp.maximum(jnp.max(sl, axis=1, keepdims=True), tl)
    s = jnp.exp(tl - m) + jnp.sum(jnp.exp(sl - m), axis=1, keepdims=True)
    out_ref[...] = m - tl + jnp.log(s)


@jax.jit
def _tc_finish(user_emb, sw, true_dot, lab2, sid_row, sb_row, corr_row):
    return pl.pallas_call(
        _tc_body,
        out_shape=jax.ShapeDtypeStruct((BATCH, 1), jnp.float32),
    )(user_emb, sw, true_dot, lab2, sid_row, sb_row, corr_row)


def kernel(item_embeddings, user_embeddings, label_idx, zero_bias):
    labels = label_idx[:, 0]
    item_flat = item_embeddings.reshape(-1)

    u = jax.random.uniform(jax.random.key(42), (NUM_SAMPLED,),
                           dtype=jnp.float32)
    ids = jnp.floor(jnp.exp(u * jnp.log(NUM_CLASSES + 1.0))) - 1.0
    sampled = jnp.clip(ids, 0, NUM_CLASSES - 1).astype(jnp.int32)
    q_sampled = (jnp.log((sampled.astype(jnp.float32) + 2.0)
                         / (sampled.astype(jnp.float32) + 1.0))
                 * _INV_LOG_RANGE)
    exp_sampled = -jnp.expm1(NUM_SAMPLED * jnp.log1p(-q_sampled))
    corr = jnp.log(exp_sampled)
    corr_row = jnp.zeros((1, S_PAD), jnp.float32).at[0, :NUM_SAMPLED].set(corr)
    sampled_pad = jnp.zeros((S_PAD,), jnp.int32).at[:NUM_SAMPLED].set(sampled)

    true_dot, sw, sb = _sc_gather(item_flat, user_embeddings, labels,
                                  sampled_pad, zero_bias)

    loss = _tc_finish(user_embeddings, sw, true_dot.reshape(BATCH, 1),
                      label_idx, sampled_pad.reshape(1, S_PAD),
                      sb.reshape(1, S_PAD), corr_row)
    return loss


# speedup vs baseline: 1.5195x; 1.5195x over previous
import functools
import math

import jax
import jax.numpy as jnp
from jax import lax
from jax.experimental import pallas as pl
from jax.experimental.pallas import tpu as pltpu
from jax.experimental.pallas import tpu_sc as plsc

NUM_SAMPLED = 100
NUM_CLASSES = 100000
DIM = 64
BATCH = 4096
S_PAD = 128
NC = 2
NS = 16
NW = NC * NS
CHUNK = BATCH // NW
L = 16

_INV_LOG_RANGE = 1.0 / math.log(NUM_CLASSES + 1.0)


def _zero_tail(ref):
    for i in range(NUM_SAMPLED // L, S_PAD // L):
        sl = pl.ds(i * L, L)
        lane = lax.broadcasted_iota(jnp.int32, (L,), 0) + i * L
        ref[sl] = jnp.where(lane < NUM_SAMPLED, ref[sl], 0.0)


def _sc_body(item_flat, user_hbm, lab_hbm, sid_hbm, bias_hbm,
             true_out, sw_out, sb_out,
             lab_v, user_v, acc_v, sid_v, sg2_v, idx2_v, g2_v, sidx_v, sg_v,
             sem_g, sem_u, sem_b, sem_s):
    wid = lax.axis_index("s") * NC + lax.axis_index("c")
    base = wid * CHUNK

    pltpu.sync_copy(lab_hbm.at[pl.ds(base, CHUNK)], lab_v)
    pltpu.async_copy(user_hbm.at[:, pl.ds(base, CHUNK)], user_v, sem_u)
    pltpu.async_copy(bias_hbm.at[lab_v], acc_v, sem_b)
    pltpu.sync_copy(sid_hbm, sid_v)

    @pl.loop(0, DIM)
    def _(d):
        off = d * NUM_CLASSES
        for i in range(CHUNK // L):
            sl = pl.ds(i * L, L)
            idx2_v[d, sl] = lab_v[sl] + off
        pltpu.async_copy(item_flat.at[idx2_v.at[d]], g2_v.at[d], sem_g)

    for r in range(DIM // NW):
        d = wid * (DIM // NW) + r
        off = d * NUM_CLASSES
        for i in range(S_PAD // L):
            sl = pl.ds(i * L, L)
            sidx_v[r, sl] = sid_v[sl] + off
        pltpu.async_copy(item_flat.at[sidx_v.at[r]], sg2_v.at[r], sem_s)

    pltpu.make_async_copy(user_hbm.at[:, pl.ds(base, CHUNK)], user_v,
                          sem_u).wait()
    pltpu.make_async_copy(bias_hbm.at[lab_v], acc_v, sem_b).wait()
    pltpu.make_async_copy(user_hbm.at[:, pl.ds(base, CHUNK)], g2_v,
                          sem_g).wait()

    def _acc_body(d, accs):
        return tuple(
            accs[i] + user_v[d, pl.ds(i * L, L)] * g2_v[d, pl.ds(i * L, L)]
            for i in range(CHUNK // L))

    accs = lax.fori_loop(
        0, DIM, _acc_body,
        tuple(acc_v[pl.ds(i * L, L)] for i in range(CHUNK // L)))
    for i in range(CHUNK // L):
        acc_v[pl.ds(i * L, L)] = accs[i]
    pltpu.sync_copy(acc_v, true_out.at[pl.ds(base, CHUNK)])

    pltpu.make_async_copy(user_hbm.at[pl.ds(0, DIM // NW), pl.ds(0, S_PAD)],
                          sg2_v, sem_s).wait()
    for r in range(DIM // NW):
        d = wid * (DIM // NW) + r
        for i in range(NUM_SAMPLED // L, S_PAD // L):
            sl = pl.ds(i * L, L)
            lane = lax.broadcasted_iota(jnp.int32, (L,), 0) + i * L
            sg2_v[r, sl] = jnp.where(lane < NUM_SAMPLED, sg2_v[r, sl], 0.0)
        pltpu.sync_copy(sg2_v.at[r], sw_out.at[d])

    @pl.when(wid == 0)
    def _():
        pltpu.sync_copy(bias_hbm.at[sid_v], sg_v)
        _zero_tail(sg_v)
        pltpu.sync_copy(sg_v, sb_out)


@jax.jit
def _sc_gather(item_flat, user_emb, labels, sampled_pad, bias):
    mesh = plsc.VectorSubcoreMesh(core_axis_name="c", subcore_axis_name="s")
    f = pl.kernel(
        _sc_body,
        out_type=(
            jax.ShapeDtypeStruct((BATCH,), jnp.float32),
            jax.ShapeDtypeStruct((DIM, S_PAD), jnp.float32),
            jax.ShapeDtypeStruct((S_PAD,), jnp.float32),
        ),
        mesh=mesh,
        scratch_types=[
            pltpu.VMEM((CHUNK,), jnp.int32),
            pltpu.VMEM((DIM, CHUNK), jnp.float32),
            pltpu.VMEM((CHUNK,), jnp.float32),
            pltpu.VMEM((S_PAD,), jnp.int32),
            pltpu.VMEM((DIM // NW, S_PAD), jnp.float32),
            pltpu.VMEM((DIM, CHUNK), jnp.int32),
            pltpu.VMEM((DIM, CHUNK), jnp.float32),
            pltpu.VMEM((DIM // NW, S_PAD), jnp.int32),
            pltpu.VMEM((S_PAD,), jnp.float32),
            pltpu.SemaphoreType.DMA,
            pltpu.SemaphoreType.DMA,
            pltpu.SemaphoreType.DMA,
            pltpu.SemaphoreType.DMA,
        ],
    )
    return f(item_flat, user_emb, labels, sampled_pad, bias)


def _tc_body(user_ref, sw_ref, td_ref, lab_ref, sid_ref, sb_ref, corr_ref,
             out_ref):
    x = user_ref[...]
    w = sw_ref[...]
    sl = lax.dot_general(x, w, (((0,), (0,)), ((), ())),
                         preferred_element_type=jnp.float32,
                         precision=lax.Precision.HIGHEST)
    sl = sl + sb_ref[...] - corr_ref[...]

    lab = lab_ref[...]
    sid = sid_ref[...]
    hits = sid == lab
    sl = jnp.where(hits, sl - 1e9, sl)
    col = lax.broadcasted_iota(jnp.int32, (1, S_PAD), 1)
    sl = jnp.where(col < NUM_SAMPLED, sl, -1e30)

    labf = lab.astype(jnp.float32)
    q_true = jnp.log((labf + 2.0) / (labf + 1.0)) * _INV_LOG_RANGE
    q = q_true
    l1p = -(q * (1.0 + q * (0.5 + q * (1.0 / 3.0 + q * (0.25 + q * 0.2)))))
    xx = NUM_SAMPLED * l1p
    small = xx > -0.2
    series = xx * (1.0 + xx * (0.5 + xx * (1.0 / 6.0 + xx * (1.0 / 24.0))))
    exp_true = -jnp.where(small, series, jnp.exp(xx) - 1.0)
    tl = td_ref[...] - jnp.log(exp_true)

    m = jnp.maximum(jnp.max(sl, axis=1, keepdims=True), tl)
    s = jnp.exp(tl - m) + jnp.sum(jnp.exp(sl - m), axis=1, keepdims=True)
    out_ref[...] = m - tl + jnp.log(s)


@jax.jit
def _tc_finish(user_emb, sw, true_dot, lab2, sid_row, sb_row, corr_row):
    return pl.pallas_call(
        _tc_body,
        out_shape=jax.ShapeDtypeStruct((BATCH, 1), jnp.float32),
    )(user_emb, sw, true_dot, lab2, sid_row, sb_row, corr_row)


def kernel(item_embeddings, user_embeddings, label_idx, zero_bias):
    labels = label_idx[:, 0]
    item_flat = item_embeddings.reshape(-1)

    u = jax.random.uniform(jax.random.key(42), (NUM_SAMPLED,),
                           dtype=jnp.float32)
    ids = jnp.floor(jnp.exp(u * jnp.log(NUM_CLASSES + 1.0))) - 1.0
    sampled = jnp.clip(ids, 0, NUM_CLASSES - 1).astype(jnp.int32)
    q_sampled = (jnp.log((sampled.astype(jnp.float32) + 2.0)
                         / (sampled.astype(jnp.float32) + 1.0))
                 * _INV_LOG_RANGE)
    exp_sampled = -jnp.expm1(NUM_SAMPLED * jnp.log1p(-q_sampled))
    corr = jnp.log(exp_sampled)
    corr_row = jnp.zeros((1, S_PAD), jnp.float32).at[0, :NUM_SAMPLED].set(corr)
    sampled_pad = jnp.zeros((S_PAD,), jnp.int32).at[:NUM_SAMPLED].set(sampled)

    true_dot, sw, sb = _sc_gather(item_flat, user_embeddings, labels,
                                  sampled_pad, zero_bias)

    loss = _tc_finish(user_embeddings, sw, true_dot.reshape(BATCH, 1),
                      label_idx, sampled_pad.reshape(1, S_PAD),
                      sb.reshape(1, S_PAD), corr_row)
    return loss
